# baseline (speedup 1.0000x reference)
import functools

import jax
import jax.numpy as jnp
from jax import lax
from jax.experimental import pallas as pl
from jax.experimental.pallas import tpu as pltpu
from jax.experimental.pallas import tpu_sc as plsc

_NC = 2
_NS = 16
_NW = _NC * _NS


def _sc_mesh():
    return plsc.VectorSubcoreMesh(core_axis_name="c", subcore_axis_name="s")


def _sc_gather(table, idx, chunk):
    B = idx.shape[0]
    D = table.shape[1]
    per_w = B // _NW
    n_chunks = per_w // chunk

    kfn = functools.partial(
        pl.kernel,
        out_type=jax.ShapeDtypeStruct((B, D), table.dtype),
        mesh=_sc_mesh(),
        scratch_types=[
            pltpu.VMEM((chunk,), jnp.int32),
            pltpu.VMEM((chunk, D), table.dtype),
            pltpu.SemaphoreType.DMA,
        ],
        compiler_params=pltpu.CompilerParams(use_tc_tiling_on_sc=False),
    )

    @kfn
    def run(table_hbm, idx_hbm, out_hbm, idx_v, rows_v, sem):
        wid = lax.axis_index("s") * _NC + lax.axis_index("c")
        base = wid * per_w

        def body(j, carry):
            o = base + j * chunk
            pltpu.sync_copy(idx_hbm.at[pl.ds(o, chunk)], idx_v)
            pltpu.async_copy(table_hbm.at[idx_v], rows_v, sem).wait()
            pltpu.sync_copy(rows_v, out_hbm.at[pl.ds(o, chunk)])
            return carry

        lax.fori_loop(0, n_chunks, body, 0)

    return run(table, idx)


def _sc_segment_sum(vals, idx2, acc_units, chunk, zeros_hbm):
    U, D = vals.shape
    per_tile = U // _NS
    n_chunks = per_tile // chunk
    upt = acc_units // _NS

    kfn = functools.partial(
        pl.kernel,
        out_type=jax.ShapeDtypeStruct((_NC, acc_units, D), jnp.float32),
        mesh=_sc_mesh(),
        scratch_types=[
            pltpu.VMEM((chunk,), jnp.int32),
            pltpu.VMEM((chunk, D), jnp.float32),
            pltpu.VMEM_SHARED((acc_units, D), jnp.float32),
        ],
        compiler_params=pltpu.CompilerParams(use_tc_tiling_on_sc=False),
    )

    @kfn
    def run(v_hbm, i_hbm, z_hbm, out_hbm, idx_v, rows_v, acc_sh):
        cid = lax.axis_index("c")
        sid = lax.axis_index("s")
        r0 = sid * upt

        pltpu.sync_copy(z_hbm.at[pl.ds(r0, upt)], acc_sh.at[pl.ds(r0, upt)])
        plsc.subcore_barrier()

        def body(j, carry):
            o = sid * per_tile + j * chunk
            pltpu.sync_copy(i_hbm.at[cid, pl.ds(o, chunk)], idx_v)
            pltpu.sync_copy(v_hbm.at[pl.ds(o, chunk)], rows_v)
            pltpu.sync_copy(rows_v, acc_sh.at[idx_v], add=True)
            return carry

        lax.fori_loop(0, n_chunks, body, 0)
        plsc.subcore_barrier()
        pltpu.sync_copy(acc_sh.at[pl.ds(r0, upt)],
                        out_hbm.at[cid, pl.ds(r0, upt)])

    return run(vals, idx2, zeros_hbm)


def _group_ind(width, gsize):
    g = width // gsize
    r = lax.broadcasted_iota(jnp.int32, (width, g), 0) // gsize
    c = lax.broadcasted_iota(jnp.int32, (width, g), 1)
    return (r == c).astype(jnp.float32)


def _ln_silu_g(y, gm, bt, gsize):
    a = _group_ind(y.shape[1], gsize)
    mu = jnp.dot(jnp.dot(y, a, preferred_element_type=jnp.float32) / gsize,
                 a.T, preferred_element_type=jnp.float32)
    c = y - mu
    var = jnp.dot(jnp.dot(c * c, a, preferred_element_type=jnp.float32) / gsize,
                  a.T, preferred_element_type=jnp.float32)
    y = c * lax.rsqrt(var + 1e-5) * gm + bt
    return y * jax.nn.sigmoid(y)


def _tc_mlp_packed(xs, specs, w1s, vec1, w2, vec2, *, grid, g1, g2,
                   out_shape, out_block, out_norm, res_idx=None, l2g=None):
    n = len(xs)

    def body(*refs):
        xr = refs[:n]
        w1r = refs[n:2 * n]
        v1r, w2r, v2r, outr = refs[2 * n:2 * n + 4]

        acc = None
        for i in range(n):
            t = jnp.dot(xr[i][...], w1r[i][...],
                        preferred_element_type=jnp.float32)
            acc = t if acc is None else acc + t
        acc = acc + v1r[0:1, :]
        acc = _ln_silu_g(acc, v1r[1:2, :], v1r[2:3, :], g1)
        y = jnp.dot(acc, w2r[...], preferred_element_type=jnp.float32)
        y = y + v2r[0:1, :]
        if out_norm:
            y = _ln_silu_g(y, v2r[1:2, :], v2r[2:3, :], g2)
        if res_idx is not None:
            y = y + xr[res_idx][...]
        if l2g is not None:
            a = _group_ind(y.shape[1], l2g)
            ss = jnp.dot(jnp.dot(y * y, a, preferred_element_type=jnp.float32),
                         a.T, preferred_element_type=jnp.float32)
            y = y / jnp.maximum(jnp.sqrt(ss), 1e-12)
        outr[...] = y

    in_arrays = list(xs) + list(w1s) + [vec1, w2, vec2]
    in_specs = [pl.BlockSpec(b, m) for (b, m) in specs]
    for warr in list(w1s) + [vec1, w2, vec2]:
        in_specs.append(pl.BlockSpec(warr.shape, lambda i: (0, 0)))

    return pl.pallas_call(
        body,
        grid=(grid,),
        in_specs=in_specs,
        out_specs=pl.BlockSpec(out_block, lambda i: (i, 0)),
        out_shape=jax.ShapeDtypeStruct(out_shape, jnp.float32),
    )(*in_arrays)


def _bd(m, k):
    di, do = m.shape
    out = jnp.zeros((di * k, do * k), m.dtype)
    for i in range(k):
        out = out.at[i * di:(i + 1) * di, i * do:(i + 1) * do].set(m)
    return out


def _vecs(layer, k):
    width = layer["b"].shape[0]
    b = jnp.tile(layer["b"], k)
    g = jnp.tile(layer.get("g", jnp.zeros((width,), jnp.float32)), k)
    be = jnp.tile(layer.get("be", jnp.zeros((width,), jnp.float32)), k)
    return jnp.stack([b, g, be], axis=0)


def kernel(x, graph, params):
    N = x.shape[0]
    E = graph.shape[1]
    L = 32
    src = graph[0]
    dst = graph[1]

    ne = params["node_encoder"]
    ee = params["edge_encoder"]
    out_p = params["output"]

    u4 = jnp.arange(4, dtype=jnp.int32)
    src4 = (src[:, None] * 4 + u4).reshape(4 * E)
    dst4 = (dst[:, None] * 4 + u4).reshape(4 * E)
    idx_cat = jnp.concatenate([src4, dst4])

    half = N // 2
    acc_rows = ((half + 512 + 1599) // 1600) * 1600
    trash = half + (jnp.arange(E, dtype=jnp.int32) % 512)
    d0 = jnp.where(dst < half, dst, trash)
    d1 = jnp.where(dst >= half, dst - half, trash)
    idx2 = jnp.stack([(d0[:, None] * 4 + u4).reshape(4 * E),
                      (d1[:, None] * 4 + u4).reshape(4 * E)])
    zeros_seg = jnp.zeros((4 * acc_rows, 8), jnp.float32)

    ep4 = E * L // 128
    np4 = N * L // 128
    row_map = lambda i: (i, 0)

    x_pack = jnp.pad(x, ((0, 0), (0, 29))).reshape(np4, 128)
    w1n = jnp.zeros((L, 64), jnp.float32).at[0:3].set(ne[0]["W"])
    nodes_p = _tc_mlp_packed(
        [x_pack], [((np4, 128), row_map)],
        [_bd(w1n, 4)], _vecs(ne[0], 4), _bd(ne[1]["W"], 4), _vecs(ne[1], 4),
        grid=1, g1=64, g2=32,
        out_shape=(np4, 128), out_block=(np4, 128), out_norm=True,
    )

    gx = _sc_gather(x_pack.reshape(4 * N, 8), idx_cat, 4000)
    gxv = gx.reshape(2 * ep4, 128)
    w1es = jnp.zeros((L, 64), jnp.float32).at[0:3].set(ee[0]["W"][0:3])
    w1ed = jnp.zeros((L, 64), jnp.float32).at[0:3].set(ee[0]["W"][3:6])
    edges_p = _tc_mlp_packed(
        [gxv, gxv],
        [((1000, 128), row_map), ((1000, 128), lambda i: (i + ep4 // 1000, 0))],
        [_bd(w1es, 4), _bd(w1ed, 4)],
        _vecs(ee[0], 4), _bd(ee[1]["W"], 4), _vecs(ee[1], 4),
        grid=ep4 // 1000, g1=64, g2=32,
        out_shape=(ep4, 128), out_block=(1000, 128), out_norm=True,
    )

    for cp in params["cells"]:
        nw = cp["node"]
        ew = cp["edge"]
        parts = _sc_segment_sum(edges_p.reshape(4 * E, 8), idx2,
                                4 * acc_rows, 4000, zeros_seg)
        v = parts.reshape(2 * acc_rows * L // 128, 128)
        region = acc_rows * L // 128
        owned = half * L // 128
        msg_p = jnp.concatenate([v[0:owned], v[region:region + owned]], axis=0)
        nodes_p = _tc_mlp_packed(
            [nodes_p, msg_p],
            [((np4, 128), row_map), ((np4, 128), row_map)],
            [_bd(nw[0]["W"][0:L], 4), _bd(nw[0]["W"][L:2 * L], 4)],
            _vecs(nw[0], 4), _bd(nw[1]["W"], 4), _vecs(nw[1], 4),
            grid=1, g1=64, g2=32,
            out_shape=(np4, 128), out_block=(np4, 128), out_norm=True,
            res_idx=0,
        )
        g = _sc_gather(nodes_p.reshape(4 * N, 8), idx_cat, 4000)
        gv = g.reshape(2 * ep4, 128)
        edges_p = _tc_mlp_packed(
            [gv, gv, edges_p],
            [((1000, 128), row_map),
             ((1000, 128), lambda i: (i + ep4 // 1000, 0)),
             ((1000, 128), row_map)],
            [_bd(ew[0]["W"][0:L], 4), _bd(ew[0]["W"][L:2 * L], 4),
             _bd(ew[0]["W"][2 * L:3 * L], 4)],
            _vecs(ew[0], 4), _bd(ew[1]["W"], 4), _vecs(ew[1], 4),
            grid=ep4 // 1000, g1=64, g2=32,
            out_shape=(ep4, 128), out_block=(1000, 128), out_norm=True,
            res_idx=2,
        )

    emb48 = _tc_mlp_packed(
        [nodes_p], [((np4, 128), row_map)],
        [_bd(out_p[0]["W"], 4)], _vecs(out_p[0], 4),
        _bd(out_p[1]["W"], 4), _vecs(out_p[1], 4),
        grid=1, g1=64, g2=12,
        out_shape=(np4, 48), out_block=(np4, 48), out_norm=False, l2g=12,
    )
    emb = emb48.reshape(N, 12)
    return emb, nodes_p.reshape(N, L), edges_p.reshape(E, L)


# speedup vs baseline: 1.3831x; 1.3831x over previous
import functools

import jax
import jax.numpy as jnp
from jax import lax
from jax.experimental import pallas as pl
from jax.experimental.pallas import tpu as pltpu
from jax.experimental.pallas import tpu_sc as plsc

_NC = 2
_NS = 16
_NW = _NC * _NS


def _sc_mesh():
    return plsc.VectorSubcoreMesh(core_axis_name="c", subcore_axis_name="s")


def _sc_gather(table, idx, chunk):
    B = idx.shape[0]
    D = table.shape[1]
    per_w = B // _NW
    n_chunks = per_w // chunk

    kfn = functools.partial(
        pl.kernel,
        out_type=jax.ShapeDtypeStruct((B, D), table.dtype),
        mesh=_sc_mesh(),
        scratch_types=[
            pltpu.VMEM((chunk,), jnp.int32),
            pltpu.VMEM((chunk, D), table.dtype),
            pltpu.SemaphoreType.DMA,
        ],
        compiler_params=pltpu.CompilerParams(use_tc_tiling_on_sc=False),
    )

    @kfn
    def run(table_hbm, idx_hbm, out_hbm, idx_v, rows_v, sem):
        wid = lax.axis_index("s") * _NC + lax.axis_index("c")
        base = wid * per_w

        def body(j, carry):
            o = base + j * chunk
            pltpu.sync_copy(idx_hbm.at[pl.ds(o, chunk)], idx_v)
            pltpu.async_copy(table_hbm.at[idx_v], rows_v, sem).wait()
            pltpu.sync_copy(rows_v, out_hbm.at[pl.ds(o, chunk)])
            return carry

        lax.fori_loop(0, n_chunks, body, 0)

    return run(table, idx)


def _sc_segment_sum(vals, idx2, acc_rows, chunk, zeros_hbm):
    E, D = vals.shape
    per_tile = E // _NS
    n_chunks = per_tile // chunk
    rows_per_tile = acc_rows // _NS

    kfn = functools.partial(
        pl.kernel,
        out_type=jax.ShapeDtypeStruct((_NC, acc_rows, D), jnp.float32),
        mesh=_sc_mesh(),
        scratch_types=[
            pltpu.VMEM((chunk,), jnp.int32),
            pltpu.VMEM((chunk, D), jnp.float32),
            pltpu.VMEM_SHARED((acc_rows, D), jnp.float32),
        ],
        compiler_params=pltpu.CompilerParams(use_tc_tiling_on_sc=False),
    )

    @kfn
    def run(v_hbm, i_hbm, z_hbm, out_hbm, idx_v, rows_v, acc_sh):
        cid = lax.axis_index("c")
        sid = lax.axis_index("s")
        r0 = sid * rows_per_tile

        pltpu.sync_copy(z_hbm.at[pl.ds(r0, rows_per_tile)],
                        acc_sh.at[pl.ds(r0, rows_per_tile)])
        plsc.subcore_barrier()

        def body(j, carry):
            o = sid * per_tile + j * chunk
            pltpu.sync_copy(i_hbm.at[cid, pl.ds(o, chunk)], idx_v)
            pltpu.sync_copy(v_hbm.at[pl.ds(o, chunk)], rows_v)
            pltpu.sync_copy(rows_v, acc_sh.at[idx_v], add=True)
            return carry

        lax.fori_loop(0, n_chunks, body, 0)
        plsc.subcore_barrier()
        pltpu.sync_copy(acc_sh.at[pl.ds(r0, rows_per_tile)],
                        out_hbm.at[cid, pl.ds(r0, rows_per_tile)])

    return run(vals, idx2, zeros_hbm)


def _group_ind(width, gsize):
    g = width // gsize
    r = lax.broadcasted_iota(jnp.int32, (width, g), 0) // gsize
    c = lax.broadcasted_iota(jnp.int32, (width, g), 1)
    return (r == c).astype(jnp.float32)


def _ln_silu_g(y, gm, bt, gsize):
    a = _group_ind(y.shape[1], gsize)
    mu = jnp.dot(jnp.dot(y, a, preferred_element_type=jnp.float32) / gsize,
                 a.T, preferred_element_type=jnp.float32)
    c = y - mu
    var = jnp.dot(jnp.dot(c * c, a, preferred_element_type=jnp.float32) / gsize,
                  a.T, preferred_element_type=jnp.float32)
    y = c * lax.rsqrt(var + 1e-5) * gm + bt
    return y * jax.nn.sigmoid(y)


def _tc_mlp_packed(xs, specs, w1s, vec1, w2, vec2, *, grid, g1, g2,
                   out_shape, out_block, out_norm, res_idx=None, l2g=None,
                   mm_bf16=False, bf16_copy=False):
    n = len(xs)
    if mm_bf16:
        w1s = [w.astype(jnp.bfloat16) for w in w1s]
        w2 = w2.astype(jnp.bfloat16)

    def body(*refs):
        xr = refs[:n]
        w1r = refs[n:2 * n]
        v1r, w2r, v2r = refs[2 * n:2 * n + 3]
        outr = refs[2 * n + 3]
        outr2 = refs[2 * n + 4] if bf16_copy else None

        acc = None
        for i in range(n):
            xv = xr[i][...]
            if mm_bf16:
                xv = xv.astype(jnp.bfloat16)
            t = jnp.dot(xv, w1r[i][...], preferred_element_type=jnp.float32)
            acc = t if acc is None else acc + t
        acc = acc + v1r[0:1, :]
        acc = _ln_silu_g(acc, v1r[1:2, :], v1r[2:3, :], g1)
        if mm_bf16:
            acc = acc.astype(jnp.bfloat16)
        y = jnp.dot(acc, w2r[...], preferred_element_type=jnp.float32)
        y = y + v2r[0:1, :]
        if out_norm:
            y = _ln_silu_g(y, v2r[1:2, :], v2r[2:3, :], g2)
        if res_idx is not None:
            y = y + xr[res_idx][...].astype(jnp.float32)
        if l2g is not None:
            a = _group_ind(y.shape[1], l2g)
            ss = jnp.dot(jnp.dot(y * y, a, preferred_element_type=jnp.float32),
                         a.T, preferred_element_type=jnp.float32)
            y = y / jnp.maximum(jnp.sqrt(ss), 1e-12)
        outr[...] = y
        if bf16_copy:
            outr2[...] = y.astype(jnp.bfloat16)

    in_arrays = list(xs) + list(w1s) + [vec1, w2, vec2]
    in_specs = [pl.BlockSpec(b, m) for (b, m) in specs]
    for warr in list(w1s) + [vec1, w2, vec2]:
        in_specs.append(pl.BlockSpec(warr.shape, lambda i: (0, 0)))

    out_shapes = jax.ShapeDtypeStruct(out_shape, jnp.float32)
    out_specs = pl.BlockSpec(out_block, lambda i: (i, 0))
    if bf16_copy:
        out_shapes = [out_shapes, jax.ShapeDtypeStruct(out_shape, jnp.bfloat16)]
        out_specs = [out_specs, pl.BlockSpec(out_block, lambda i: (i, 0))]

    return pl.pallas_call(
        body,
        grid=(grid,),
        in_specs=in_specs,
        out_specs=out_specs,
        out_shape=out_shapes,
    )(*in_arrays)


def _blk(rows, want):
    return want if rows % want == 0 else rows


def _bd(m, k):
    di, do = m.shape
    out = jnp.zeros((di * k, do * k), m.dtype)
    for i in range(k):
        out = out.at[i * di:(i + 1) * di, i * do:(i + 1) * do].set(m)
    return out


def _vecs(layer, k):
    width = layer["b"].shape[0]
    b = jnp.tile(layer["b"], k)
    g = jnp.tile(layer.get("g", jnp.zeros((width,), jnp.float32)), k)
    be = jnp.tile(layer.get("be", jnp.zeros((width,), jnp.float32)), k)
    return jnp.stack([b, g, be], axis=0)


def kernel(x, graph, params):
    N = x.shape[0]
    E = graph.shape[1]
    L = 32
    src = graph[0]
    dst = graph[1]

    ne = params["node_encoder"]
    ee = params["edge_encoder"]
    out_p = params["output"]

    half = N // 2
    acc_rows = ((half + 512 + 1599) // 1600) * 1600
    trash = half + (jnp.arange(E, dtype=jnp.int32) % 512)
    idx2 = jnp.stack([jnp.where(dst < half, dst, trash),
                      jnp.where(dst >= half, dst - half, trash)])
    zeros_seg = jnp.zeros((acc_rows, L), jnp.float32)

    ep4 = E * L // 128
    np4 = N * L // 128
    row_map = lambda i: (i, 0)

    x_pack = jnp.pad(x, ((0, 0), (0, 29))).reshape(np4, 128)
    w1n = jnp.zeros((L, 64), jnp.float32).at[0:3].set(ne[0]["W"])
    nodes_p, nodes_bf = _tc_mlp_packed(
        [x_pack], [((np4, 128), row_map)],
        [_bd(w1n, 4)], _vecs(ne[0], 4), _bd(ne[1]["W"], 4), _vecs(ne[1], 4),
        grid=1, g1=64, g2=32,
        out_shape=(np4, 128), out_block=(np4, 128), out_norm=True,
        bf16_copy=True,
    )

    x8 = jnp.pad(x, ((0, 0), (0, 5)))
    gxs = _sc_gather(x8, src, 1000).reshape(E * 8 // 128, 128)
    gxd = _sc_gather(x8, dst, 1000).reshape(E * 8 // 128, 128)
    w1es = jnp.zeros((8, 64), jnp.float32).at[0:3].set(ee[0]["W"][0:3])
    w1ed = jnp.zeros((8, 64), jnp.float32).at[0:3].set(ee[0]["W"][3:6])
    rows8 = E * 8 // 128
    b8 = _blk(rows8, 1000)
    enc = _tc_mlp_packed(
        [gxs, gxd],
        [((b8, 128), row_map), ((b8, 128), row_map)],
        [_bd(w1es, 16), _bd(w1ed, 16)],
        _vecs(ee[0], 16), _bd(ee[1]["W"], 16), _vecs(ee[1], 16),
        grid=rows8 // b8, g1=64, g2=32,
        out_shape=(rows8, 512), out_block=(b8, 512), out_norm=True,
        mm_bf16=True,
    )
    edges_p = enc.reshape(ep4, 128)

    for cp in params["cells"]:
        nw = cp["node"]
        ew = cp["edge"]
        parts = _sc_segment_sum(edges_p.reshape(E, L), idx2, acc_rows,
                                1000, zeros_seg)
        v = parts.reshape(2 * acc_rows * L // 128, 128)
        region = acc_rows * L // 128
        owned = half * L // 128
        msg_p = jnp.concatenate([v[0:owned], v[region:region + owned]], axis=0)
        nodes_p, nodes_bf = _tc_mlp_packed(
            [nodes_p, msg_p],
            [((np4, 128), row_map), ((np4, 128), row_map)],
            [_bd(nw[0]["W"][0:L], 4), _bd(nw[0]["W"][L:2 * L], 4)],
            _vecs(nw[0], 4), _bd(nw[1]["W"], 4), _vecs(nw[1], 4),
            grid=1, g1=64, g2=32,
            out_shape=(np4, 128), out_block=(np4, 128), out_norm=True,
            res_idx=0, bf16_copy=True,
        )
        gs = _sc_gather(nodes_bf.reshape(N, L), src, 1000).reshape(ep4, 128)
        gd = _sc_gather(nodes_bf.reshape(N, L), dst, 1000).reshape(ep4, 128)
        be = _blk(ep4, 2000)
        edges_p = _tc_mlp_packed(
            [gs, gd, edges_p],
            [((be, 128), row_map), ((be, 128), row_map),
             ((be, 128), row_map)],
            [_bd(ew[0]["W"][0:L], 4), _bd(ew[0]["W"][L:2 * L], 4),
             _bd(ew[0]["W"][2 * L:3 * L], 4)],
            _vecs(ew[0], 4), _bd(ew[1]["W"], 4), _vecs(ew[1], 4),
            grid=ep4 // be, g1=64, g2=32,
            out_shape=(ep4, 128), out_block=(be, 128), out_norm=True,
            res_idx=2, mm_bf16=True,
        )

    emb48 = _tc_mlp_packed(
        [nodes_p], [((np4, 128), row_map)],
        [_bd(out_p[0]["W"], 4)], _vecs(out_p[0], 4),
        _bd(out_p[1]["W"], 4), _vecs(out_p[1], 4),
        grid=1, g1=64, g2=12,
        out_shape=(np4, 48), out_block=(np4, 48), out_norm=False, l2g=12,
    )
    emb = emb48.reshape(N, 12)
    return emb, nodes_p.reshape(N, L), edges_p.reshape(E, L)


# speedup vs baseline: 1.4680x; 1.0614x over previous
import functools

import jax
import jax.numpy as jnp
from jax import lax
from jax.experimental import pallas as pl
from jax.experimental.pallas import tpu as pltpu
from jax.experimental.pallas import tpu_sc as plsc

_NC = 2
_NS = 16
_NW = _NC * _NS


def _sc_mesh():
    return plsc.VectorSubcoreMesh(core_axis_name="c", subcore_axis_name="s")


def _sc_gather(table, idx, chunk):
    B = idx.shape[0]
    D = table.shape[1]
    per_w = B // _NW
    n_chunks = per_w // chunk

    kfn = functools.partial(
        pl.kernel,
        out_type=jax.ShapeDtypeStruct((B, D), table.dtype),
        mesh=_sc_mesh(),
        scratch_types=[
            pltpu.VMEM((chunk,), jnp.int32),
            pltpu.VMEM((chunk, D), table.dtype),
            pltpu.SemaphoreType.DMA,
        ],
        compiler_params=pltpu.CompilerParams(use_tc_tiling_on_sc=False),
    )

    @kfn
    def run(table_hbm, idx_hbm, out_hbm, idx_v, rows_v, sem):
        wid = lax.axis_index("s") * _NC + lax.axis_index("c")
        base = wid * per_w

        def body(j, carry):
            o = base + j * chunk
            pltpu.sync_copy(idx_hbm.at[pl.ds(o, chunk)], idx_v)
            pltpu.async_copy(table_hbm.at[idx_v], rows_v, sem).wait()
            pltpu.sync_copy(rows_v, out_hbm.at[pl.ds(o, chunk)])
            return carry

        lax.fori_loop(0, n_chunks, body, 0)

    return run(table, idx)


def _sc_segment_sum(vals, idx2, acc_rows, chunk, zeros_hbm):
    E, D = vals.shape
    per_tile = E // _NS
    n_chunks = per_tile // chunk
    rows_per_tile = acc_rows // _NS

    kfn = functools.partial(
        pl.kernel,
        out_type=jax.ShapeDtypeStruct((_NC, acc_rows, D), jnp.float32),
        mesh=_sc_mesh(),
        scratch_types=[
            pltpu.VMEM((chunk,), jnp.int32),
            pltpu.VMEM((chunk, D), jnp.float32),
            pltpu.VMEM_SHARED((acc_rows, D), jnp.float32),
        ],
        compiler_params=pltpu.CompilerParams(use_tc_tiling_on_sc=False),
    )

    @kfn
    def run(v_hbm, i_hbm, z_hbm, out_hbm, idx_v, rows_v, acc_sh):
        cid = lax.axis_index("c")
        sid = lax.axis_index("s")
        r0 = sid * rows_per_tile

        pltpu.sync_copy(z_hbm.at[pl.ds(r0, rows_per_tile)],
                        acc_sh.at[pl.ds(r0, rows_per_tile)])
        plsc.subcore_barrier()

        def body(j, carry):
            o = sid * per_tile + j * chunk
            pltpu.sync_copy(i_hbm.at[cid, pl.ds(o, chunk)], idx_v)
            pltpu.sync_copy(v_hbm.at[pl.ds(o, chunk)], rows_v)
            pltpu.sync_copy(rows_v, acc_sh.at[idx_v], add=True)
            return carry

        lax.fori_loop(0, n_chunks, body, 0)
        plsc.subcore_barrier()
        pltpu.sync_copy(acc_sh.at[pl.ds(r0, rows_per_tile)],
                        out_hbm.at[cid, pl.ds(r0, rows_per_tile)])

    return run(vals, idx2, zeros_hbm)


def _group_ind(width, gsize):
    g = width // gsize
    r = lax.broadcasted_iota(jnp.int32, (width, g), 0) // gsize
    c = lax.broadcasted_iota(jnp.int32, (width, g), 1)
    return (r == c).astype(jnp.float32)


def _ln_silu_g(y, gm, bt, gsize):
    a = _group_ind(y.shape[1], gsize)
    mu = jnp.dot(jnp.dot(y, a, preferred_element_type=jnp.float32) / gsize,
                 a.T, preferred_element_type=jnp.float32)
    c = y - mu
    var = jnp.dot(jnp.dot(c * c, a, preferred_element_type=jnp.float32) / gsize,
                  a.T, preferred_element_type=jnp.float32)
    y = c * lax.rsqrt(var + 1e-5) * gm + bt
    return y * jax.nn.sigmoid(y)


def _tc_mlp_packed(xs, specs, w1s, vec1, w2, vec2, *, grid, g1, g2,
                   out_shape, out_block, out_norm, res_idx=None, l2g=None,
                   mm_bf16=False, bf16_copy=False):
    n = len(xs)
    if mm_bf16 is True or mm_bf16 is False:
        mm_parts = [mm_bf16] * n
        mm_w2 = mm_bf16
    else:
        mm_parts = list(mm_bf16)
        mm_w2 = False
    w1s = [w.astype(jnp.bfloat16) if p else w for w, p in zip(w1s, mm_parts)]
    if mm_w2:
        w2 = w2.astype(jnp.bfloat16)

    def body(*refs):
        xr = refs[:n]
        w1r = refs[n:2 * n]
        v1r, w2r, v2r = refs[2 * n:2 * n + 3]
        outr = refs[2 * n + 3]
        outr2 = refs[2 * n + 4] if bf16_copy else None

        acc = None
        for i in range(n):
            xv = xr[i][...]
            if mm_parts[i]:
                xv = xv.astype(jnp.bfloat16)
            t = jnp.dot(xv, w1r[i][...], preferred_element_type=jnp.float32)
            acc = t if acc is None else acc + t
        acc = acc + v1r[0:1, :]
        acc = _ln_silu_g(acc, v1r[1:2, :], v1r[2:3, :], g1)
        if mm_w2:
            acc = acc.astype(jnp.bfloat16)
        y = jnp.dot(acc, w2r[...], preferred_element_type=jnp.float32)
        y = y + v2r[0:1, :]
        if out_norm:
            y = _ln_silu_g(y, v2r[1:2, :], v2r[2:3, :], g2)
        if res_idx is not None:
            y = y + xr[res_idx][...].astype(jnp.float32)
        if l2g is not None:
            a = _group_ind(y.shape[1], l2g)
            ss = jnp.dot(jnp.dot(y * y, a, preferred_element_type=jnp.float32),
                         a.T, preferred_element_type=jnp.float32)
            y = y / jnp.maximum(jnp.sqrt(ss), 1e-12)
        outr[...] = y
        if bf16_copy:
            outr2[...] = y.astype(jnp.bfloat16)

    in_arrays = list(xs) + list(w1s) + [vec1, w2, vec2]
    in_specs = [pl.BlockSpec(b, m) for (b, m) in specs]
    for warr in list(w1s) + [vec1, w2, vec2]:
        in_specs.append(pl.BlockSpec(warr.shape, lambda i: (0, 0)))

    out_shapes = jax.ShapeDtypeStruct(out_shape, jnp.float32)
    out_specs = pl.BlockSpec(out_block, lambda i: (i, 0))
    if bf16_copy:
        out_shapes = [out_shapes, jax.ShapeDtypeStruct(out_shape, jnp.bfloat16)]
        out_specs = [out_specs, pl.BlockSpec(out_block, lambda i: (i, 0))]

    return pl.pallas_call(
        body,
        grid=(grid,),
        in_specs=in_specs,
        out_specs=out_specs,
        out_shape=out_shapes,
    )(*in_arrays)


def _blk(rows, want):
    return want if rows % want == 0 else rows


def _bd(m, k):
    di, do = m.shape
    out = jnp.zeros((di * k, do * k), m.dtype)
    for i in range(k):
        out = out.at[i * di:(i + 1) * di, i * do:(i + 1) * do].set(m)
    return out


def _vecs(layer, k):
    width = layer["b"].shape[0]
    b = jnp.tile(layer["b"], k)
    g = jnp.tile(layer.get("g", jnp.zeros((width,), jnp.float32)), k)
    be = jnp.tile(layer.get("be", jnp.zeros((width,), jnp.float32)), k)
    return jnp.stack([b, g, be], axis=0)


def kernel(x, graph, params):
    N = x.shape[0]
    E = graph.shape[1]
    L = 32
    src = graph[0]
    dst = graph[1]

    ne = params["node_encoder"]
    ee = params["edge_encoder"]
    out_p = params["output"]

    half = N // 2
    acc_rows = ((half + 512 + 1599) // 1600) * 1600
    trash = half + (jnp.arange(E, dtype=jnp.int32) % 512)
    idx2 = jnp.stack([jnp.where(dst < half, dst, trash),
                      jnp.where(dst >= half, dst - half, trash)])
    zeros_seg = jnp.zeros((acc_rows, L), jnp.float32)

    ep4 = E * L // 128
    np4 = N * L // 128
    row_map = lambda i: (i, 0)

    x_pack = jnp.pad(x, ((0, 0), (0, 29))).reshape(np4, 128)
    w1n = jnp.zeros((L, 64), jnp.float32).at[0:3].set(ne[0]["W"])
    nodes_p, nodes_bf = _tc_mlp_packed(
        [x_pack], [((np4, 128), row_map)],
        [_bd(w1n, 4)], _vecs(ne[0], 4), _bd(ne[1]["W"], 4), _vecs(ne[1], 4),
        grid=1, g1=64, g2=32,
        out_shape=(np4, 128), out_block=(np4, 128), out_norm=True,
        bf16_copy=True,
    )

    x8 = jnp.pad(x, ((0, 0), (0, 5)))
    gxs = _sc_gather(x8, src, 1000).reshape(E * 8 // 128, 128)
    gxd = _sc_gather(x8, dst, 1000).reshape(E * 8 // 128, 128)
    w1es = jnp.zeros((8, 64), jnp.float32).at[0:3].set(ee[0]["W"][0:3])
    w1ed = jnp.zeros((8, 64), jnp.float32).at[0:3].set(ee[0]["W"][3:6])
    rows8 = E * 8 // 128
    b8 = _blk(rows8, 1000)
    enc = _tc_mlp_packed(
        [gxs, gxd],
        [((b8, 128), row_map), ((b8, 128), row_map)],
        [_bd(w1es, 16), _bd(w1ed, 16)],
        _vecs(ee[0], 16), _bd(ee[1]["W"], 16), _vecs(ee[1], 16),
        grid=rows8 // b8, g1=64, g2=32,
        out_shape=(rows8, 512), out_block=(b8, 512), out_norm=True,
    )
    edges_p = enc.reshape(ep4, 128)

    for cp in params["cells"]:
        nw = cp["node"]
        ew = cp["edge"]
        parts = _sc_segment_sum(edges_p.reshape(E, L), idx2, acc_rows,
                                1000, zeros_seg)
        v = parts.reshape(2 * acc_rows * L // 128, 128)
        region = acc_rows * L // 128
        owned = half * L // 128
        msg_p = jnp.concatenate([v[0:owned], v[region:region + owned]], axis=0)
        nodes_p, nodes_bf = _tc_mlp_packed(
            [nodes_p, msg_p],
            [((np4, 128), row_map), ((np4, 128), row_map)],
            [_bd(nw[0]["W"][0:L], 4), _bd(nw[0]["W"][L:2 * L], 4)],
            _vecs(nw[0], 4), _bd(nw[1]["W"], 4), _vecs(nw[1], 4),
            grid=1, g1=64, g2=32,
            out_shape=(np4, 128), out_block=(np4, 128), out_norm=True,
            res_idx=0, bf16_copy=True,
        )
        gs = _sc_gather(nodes_bf.reshape(N, L), src, 1000).reshape(ep4, 128)
        gd = _sc_gather(nodes_bf.reshape(N, L), dst, 1000).reshape(ep4, 128)
        be = _blk(ep4, 2000)
        edges_p = _tc_mlp_packed(
            [gs, gd, edges_p],
            [((be, 128), row_map), ((be, 128), row_map),
             ((be, 128), row_map)],
            [_bd(ew[0]["W"][0:L], 4), _bd(ew[0]["W"][L:2 * L], 4),
             _bd(ew[0]["W"][2 * L:3 * L], 4)],
            _vecs(ew[0], 4), _bd(ew[1]["W"], 4), _vecs(ew[1], 4),
            grid=ep4 // be, g1=64, g2=32,
            out_shape=(ep4, 128), out_block=(be, 128), out_norm=True,
            res_idx=2, mm_bf16=[True, True, False],
        )

    emb48 = _tc_mlp_packed(
        [nodes_p], [((np4, 128), row_map)],
        [_bd(out_p[0]["W"], 4)], _vecs(out_p[0], 4),
        _bd(out_p[1]["W"], 4), _vecs(out_p[1], 4),
        grid=1, g1=64, g2=12,
        out_shape=(np4, 48), out_block=(np4, 48), out_norm=False, l2g=12,
    )
    emb = emb48.reshape(N, 12)
    return emb, nodes_p.reshape(N, L), edges_p.reshape(E, L)


# speedup vs baseline: 1.5234x; 1.0377x over previous
import functools

import jax
import jax.numpy as jnp
from jax import lax
from jax.experimental import pallas as pl
from jax.experimental.pallas import tpu as pltpu
from jax.experimental.pallas import tpu_sc as plsc

_NC = 2
_NS = 16
_NW = _NC * _NS


def _sc_mesh():
    return plsc.VectorSubcoreMesh(core_axis_name="c", subcore_axis_name="s")


def _sc_gather(table, idx, chunk):
    B = idx.shape[0]
    D = table.shape[1]
    per_w = B // _NW
    n_chunks = per_w // chunk

    kfn = functools.partial(
        pl.kernel,
        out_type=jax.ShapeDtypeStruct((B, D), table.dtype),
        mesh=_sc_mesh(),
        scratch_types=[
            pltpu.VMEM((chunk,), jnp.int32),
            pltpu.VMEM((chunk, D), table.dtype),
            pltpu.SemaphoreType.DMA,
        ],
        compiler_params=pltpu.CompilerParams(use_tc_tiling_on_sc=False),
    )

    @kfn
    def run(table_hbm, idx_hbm, out_hbm, idx_v, rows_v, sem):
        wid = lax.axis_index("s") * _NC + lax.axis_index("c")
        base = wid * per_w

        def body(j, carry):
            o = base + j * chunk
            pltpu.sync_copy(idx_hbm.at[pl.ds(o, chunk)], idx_v)
            pltpu.async_copy(table_hbm.at[idx_v], rows_v, sem).wait()
            pltpu.sync_copy(rows_v, out_hbm.at[pl.ds(o, chunk)])
            return carry

        lax.fori_loop(0, n_chunks, body, 0)

    return run(table, idx)


def _sc_segment_sum(vals, idx2, acc_rows, chunk, zeros_hbm):
    E, D = vals.shape
    per_tile = E // _NS
    n_chunks = per_tile // chunk
    rows_per_tile = acc_rows // _NS

    kfn = functools.partial(
        pl.kernel,
        out_type=jax.ShapeDtypeStruct((_NC, acc_rows, D), jnp.float32),
        mesh=_sc_mesh(),
        scratch_types=[
            pltpu.VMEM((chunk,), jnp.int32),
            pltpu.VMEM((chunk, D), jnp.float32),
            pltpu.VMEM_SHARED((acc_rows, D), jnp.float32),
        ],
        compiler_params=pltpu.CompilerParams(use_tc_tiling_on_sc=False),
    )

    @kfn
    def run(v_hbm, i_hbm, z_hbm, out_hbm, idx_v, rows_v, acc_sh):
        cid = lax.axis_index("c")
        sid = lax.axis_index("s")
        r0 = sid * rows_per_tile

        pltpu.sync_copy(z_hbm.at[pl.ds(r0, rows_per_tile)],
                        acc_sh.at[pl.ds(r0, rows_per_tile)])
        plsc.subcore_barrier()

        def body(j, carry):
            o = sid * per_tile + j * chunk
            pltpu.sync_copy(i_hbm.at[cid, pl.ds(o, chunk)], idx_v)
            pltpu.sync_copy(v_hbm.at[pl.ds(o, chunk)], rows_v)
            pltpu.sync_copy(rows_v, acc_sh.at[idx_v], add=True)
            return carry

        lax.fori_loop(0, n_chunks, body, 0)
        plsc.subcore_barrier()
        pltpu.sync_copy(acc_sh.at[pl.ds(r0, rows_per_tile)],
                        out_hbm.at[cid, pl.ds(r0, rows_per_tile)])

    return run(vals, idx2, zeros_hbm)


def _group_ind(width, gsize):
    g = width // gsize
    r = lax.broadcasted_iota(jnp.int32, (width, g), 0) // gsize
    c = lax.broadcasted_iota(jnp.int32, (width, g), 1)
    return (r == c).astype(jnp.float32)


def _ln_silu_g(y, gm, bt, gsize):
    a = _group_ind(y.shape[1], gsize)
    mu = jnp.dot(jnp.dot(y, a, preferred_element_type=jnp.float32) / gsize,
                 a.T, preferred_element_type=jnp.float32)
    c = y - mu
    var = jnp.dot(jnp.dot(c * c, a, preferred_element_type=jnp.float32) / gsize,
                  a.T, preferred_element_type=jnp.float32)
    y = c * lax.rsqrt(var + 1e-5) * gm + bt
    return y * jax.nn.sigmoid(y)


def _tc_mlp_packed(xs, specs, w1s, vec1, w2, vec2, *, grid, g1, g2,
                   out_shape, out_block, out_norm, res_idx=None, l2g=None,
                   mm_bf16=False, bf16_copy=False):
    n = len(xs)
    if mm_bf16 is True or mm_bf16 is False:
        mm_parts = [mm_bf16] * n
        mm_w2 = mm_bf16
    else:
        mm_parts = list(mm_bf16)
        mm_w2 = False
    w1s = [w.astype(jnp.bfloat16) if p else w for w, p in zip(w1s, mm_parts)]
    if mm_w2:
        w2 = w2.astype(jnp.bfloat16)

    def body(*refs):
        xr = refs[:n]
        w1r = refs[n:2 * n]
        v1r, w2r, v2r = refs[2 * n:2 * n + 3]
        outr = refs[2 * n + 3]
        outr2 = refs[2 * n + 4] if bf16_copy else None

        acc = None
        for i in range(n):
            xv = xr[i][...]
            if mm_parts[i]:
                xv = xv.astype(jnp.bfloat16)
            t = jnp.dot(xv, w1r[i][...], preferred_element_type=jnp.float32)
            acc = t if acc is None else acc + t
        acc = acc + v1r[0:1, :]
        acc = _ln_silu_g(acc, v1r[1:2, :], v1r[2:3, :], g1)
        if mm_w2:
            acc = acc.astype(jnp.bfloat16)
        y = jnp.dot(acc, w2r[...], preferred_element_type=jnp.float32)
        y = y + v2r[0:1, :]
        if out_norm:
            y = _ln_silu_g(y, v2r[1:2, :], v2r[2:3, :], g2)
        if res_idx is not None:
            y = y + xr[res_idx][...].astype(jnp.float32)
        if l2g is not None:
            a = _group_ind(y.shape[1], l2g)
            ss = jnp.dot(jnp.dot(y * y, a, preferred_element_type=jnp.float32),
                         a.T, preferred_element_type=jnp.float32)
            y = y / jnp.maximum(jnp.sqrt(ss), 1e-12)
        outr[...] = y
        if bf16_copy:
            outr2[...] = y.astype(jnp.bfloat16)

    in_arrays = list(xs) + list(w1s) + [vec1, w2, vec2]
    in_specs = [pl.BlockSpec(b, m) for (b, m) in specs]
    for warr in list(w1s) + [vec1, w2, vec2]:
        in_specs.append(pl.BlockSpec(warr.shape, lambda i: (0, 0)))

    out_shapes = jax.ShapeDtypeStruct(out_shape, jnp.float32)
    out_specs = pl.BlockSpec(out_block, lambda i: (i, 0))
    if bf16_copy:
        out_shapes = [out_shapes, jax.ShapeDtypeStruct(out_shape, jnp.bfloat16)]
        out_specs = [out_specs, pl.BlockSpec(out_block, lambda i: (i, 0))]

    return pl.pallas_call(
        body,
        grid=(grid,),
        in_specs=in_specs,
        out_specs=out_specs,
        out_shape=out_shapes,
    )(*in_arrays)


def _blk(rows, want):
    return want if rows % want == 0 else rows


def _bd(m, k):
    di, do = m.shape
    out = jnp.zeros((di * k, do * k), m.dtype)
    for i in range(k):
        out = out.at[i * di:(i + 1) * di, i * do:(i + 1) * do].set(m)
    return out


def _vecs(layer, k):
    width = layer["b"].shape[0]
    b = jnp.tile(layer["b"], k)
    g = jnp.tile(layer.get("g", jnp.zeros((width,), jnp.float32)), k)
    be = jnp.tile(layer.get("be", jnp.zeros((width,), jnp.float32)), k)
    return jnp.stack([b, g, be], axis=0)


def kernel(x, graph, params):
    N = x.shape[0]
    E = graph.shape[1]
    L = 32
    src = graph[0]
    dst = graph[1]

    ne = params["node_encoder"]
    ee = params["edge_encoder"]
    out_p = params["output"]

    half = N // 2
    acc_rows = ((half + 512 + 1599) // 1600) * 1600
    trash = half + (jnp.arange(E, dtype=jnp.int32) % 512)
    idx2 = jnp.stack([jnp.where(dst < half, dst, trash),
                      jnp.where(dst >= half, dst - half, trash)])
    zeros_seg = jnp.zeros((acc_rows, L), jnp.float32)

    ep4 = E * L // 128
    np4 = N * L // 128
    row_map = lambda i: (i, 0)

    x_pack = jnp.pad(x, ((0, 0), (0, 29))).reshape(np4, 128)
    w1n = jnp.zeros((L, 64), jnp.float32).at[0:3].set(ne[0]["W"])
    nodes_p, nodes_bf = _tc_mlp_packed(
        [x_pack], [((np4, 128), row_map)],
        [_bd(w1n, 4)], _vecs(ne[0], 4), _bd(ne[1]["W"], 4), _vecs(ne[1], 4),
        grid=1, g1=64, g2=32,
        out_shape=(np4, 128), out_block=(np4, 128), out_norm=True,
        bf16_copy=True,
    )

    x8 = jnp.pad(x, ((0, 0), (0, 5)))
    gxs = _sc_gather(x8, src, 5000).reshape(E * 8 // 128, 128)
    gxd = _sc_gather(x8, dst, 5000).reshape(E * 8 // 128, 128)
    w1es = jnp.zeros((8, 64), jnp.float32).at[0:3].set(ee[0]["W"][0:3])
    w1ed = jnp.zeros((8, 64), jnp.float32).at[0:3].set(ee[0]["W"][3:6])
    rows8 = E * 8 // 128
    b8 = _blk(rows8, 1000)
    enc = _tc_mlp_packed(
        [gxs, gxd],
        [((b8, 128), row_map), ((b8, 128), row_map)],
        [_bd(w1es, 16), _bd(w1ed, 16)],
        _vecs(ee[0], 16), _bd(ee[1]["W"], 16), _vecs(ee[1], 16),
        grid=rows8 // b8, g1=64, g2=32,
        out_shape=(rows8, 512), out_block=(b8, 512), out_norm=True,
    )
    edges_p = enc.reshape(ep4, 128)

    for cp in params["cells"]:
        nw = cp["node"]
        ew = cp["edge"]
        parts = _sc_segment_sum(edges_p.reshape(E, L), idx2, acc_rows,
                                2000, zeros_seg)
        v = parts.reshape(2 * acc_rows * L // 128, 128)
        region = acc_rows * L // 128
        owned = half * L // 128
        msg_p = jnp.concatenate([v[0:owned], v[region:region + owned]], axis=0)
        nodes_p, nodes_bf = _tc_mlp_packed(
            [nodes_p, msg_p],
            [((np4, 128), row_map), ((np4, 128), row_map)],
            [_bd(nw[0]["W"][0:L], 4), _bd(nw[0]["W"][L:2 * L], 4)],
            _vecs(nw[0], 4), _bd(nw[1]["W"], 4), _vecs(nw[1], 4),
            grid=1, g1=64, g2=32,
            out_shape=(np4, 128), out_block=(np4, 128), out_norm=True,
            res_idx=0, bf16_copy=True,
        )
        gs = _sc_gather(nodes_bf.reshape(N, L), src, 5000).reshape(ep4, 128)
        gd = _sc_gather(nodes_bf.reshape(N, L), dst, 5000).reshape(ep4, 128)
        be = _blk(ep4, 2000)
        edges_p = _tc_mlp_packed(
            [gs, gd, edges_p],
            [((be, 128), row_map), ((be, 128), row_map),
             ((be, 128), row_map)],
            [_bd(ew[0]["W"][0:L], 4), _bd(ew[0]["W"][L:2 * L], 4),
             _bd(ew[0]["W"][2 * L:3 * L], 4)],
            _vecs(ew[0], 4), _bd(ew[1]["W"], 4), _vecs(ew[1], 4),
            grid=ep4 // be, g1=64, g2=32,
            out_shape=(ep4, 128), out_block=(be, 128), out_norm=True,
            res_idx=2, mm_bf16=[True, True, False],
        )

    emb48 = _tc_mlp_packed(
        [nodes_p], [((np4, 128), row_map)],
        [_bd(out_p[0]["W"], 4)], _vecs(out_p[0], 4),
        _bd(out_p[1]["W"], 4), _vecs(out_p[1], 4),
        grid=1, g1=64, g2=12,
        out_shape=(np4, 48), out_block=(np4, 48), out_norm=False, l2g=12,
    )
    emb = emb48.reshape(N, 12)
    return emb, nodes_p.reshape(N, L), edges_p.reshape(E, L)
